# Initial kernel scaffold; baseline (speedup 1.0000x reference)
import jax
import jax.numpy as jnp
from jax import lax
from jax.experimental import pallas as pl
from jax.experimental.pallas import tpu as pltpu
from jax.experimental.pallas import tpu_sc as plsc

N = 10000
D = 128
NE = 320000
NCLS = 40
NC = 2
NS = 16
NW = NC * NS
N_EXT = 10240
EPT = NE // NW + 240
CH = 128
NCHUNK = EPT // CH
RPT = N_EXT // NS

_mesh = plsc.VectorSubcoreMesh(
    core_axis_name="c", subcore_axis_name="s", num_cores=NC, num_subcores=NS
)


def _deg_body(dst_hbm, out_hbm, idx_v, ones_v, zbuf, degacc):
    cid = lax.axis_index("c")
    sid = lax.axis_index("s")
    wid = cid * NS + sid

    @pl.loop(0, CH)
    def _(r):
        ones_v[r, :] = jnp.ones((16,), jnp.float32)

    @pl.loop(0, RPT)
    def _(r):
        zbuf[r, :] = jnp.zeros((16,), jnp.float32)

    pltpu.sync_copy(zbuf, degacc.at[pl.ds(sid * RPT, RPT)])
    plsc.subcore_barrier()

    @pl.loop(0, NCHUNK)
    def _(c):
        pltpu.sync_copy(dst_hbm.at[wid, pl.ds(c * CH, CH)], idx_v)
        pltpu.sync_copy(ones_v, degacc.at[idx_v], add=True)

    plsc.subcore_barrier()
    pltpu.sync_copy(
        degacc.at[pl.ds(sid * RPT, RPT)], out_hbm.at[cid, pl.ds(sid * RPT, RPT)]
    )


def _deg_sc(dst):
    return pl.kernel(
        _deg_body,
        out_type=jax.ShapeDtypeStruct((NC, N_EXT, 16), jnp.float32),
        mesh=_mesh,
        scratch_types=[
            pltpu.VMEM((CH,), jnp.int32),
            pltpu.VMEM((CH, 16), jnp.float32),
            pltpu.VMEM((RPT, 16), jnp.float32),
            pltpu.VMEM_SHARED((N_EXT, 16), jnp.float32),
        ],
    )(dst)


def _hop_body(g_hbm, src_hbm, dst_hbm, out_hbm, sidx, didx, rows, zbuf, acc):
    cid = lax.axis_index("c")
    sid = lax.axis_index("s")
    wid = cid * NS + sid

    @pl.loop(0, CH)
    def _(r):
        @pl.loop(0, D, step=16)
        def _(c):
            zbuf[r, pl.ds(c, 16)] = jnp.zeros((16,), jnp.float32)

    @pl.loop(0, RPT // CH)
    def _(j):
        pltpu.sync_copy(zbuf, acc.at[pl.ds(sid * RPT + j * CH, CH)])

    plsc.subcore_barrier()

    @pl.loop(0, NCHUNK)
    def _(c):
        pltpu.sync_copy(src_hbm.at[wid, pl.ds(c * CH, CH)], sidx)
        pltpu.sync_copy(g_hbm.at[sidx], rows)
        pltpu.sync_copy(dst_hbm.at[wid, pl.ds(c * CH, CH)], didx)
        pltpu.sync_copy(rows, acc.at[didx], add=True)

    plsc.subcore_barrier()
    pltpu.sync_copy(
        acc.at[pl.ds(sid * RPT, RPT)], out_hbm.at[cid, pl.ds(sid * RPT, RPT)]
    )


def _hop_sc(g, src, dst):
    return pl.kernel(
        _hop_body,
        out_type=jax.ShapeDtypeStruct((NC, N_EXT, D), jnp.float32),
        mesh=_mesh,
        scratch_types=[
            pltpu.VMEM((CH,), jnp.int32),
            pltpu.VMEM((CH,), jnp.int32),
            pltpu.VMEM((CH, D), jnp.float32),
            pltpu.VMEM((CH, D), jnp.float32),
            pltpu.VMEM_SHARED((N_EXT, D), jnp.float32),
        ],
    )(g, src, dst)


_RB = 1024


def _lin1_kernel(x_ref, w_ref, b_ref, o_ref):
    o_ref[...] = (
        jnp.dot(
            x_ref[...],
            w_ref[...],
            preferred_element_type=jnp.float32,
            precision=lax.Precision.HIGHEST,
        )
        + b_ref[...]
    )


def _lin1(x, W1, b1):
    return pl.pallas_call(
        _lin1_kernel,
        grid=(10,),
        in_specs=[
            pl.BlockSpec((1000, D), lambda i: (i, 0)),
            pl.BlockSpec((D, D), lambda i: (0, 0)),
            pl.BlockSpec((1, D), lambda i: (0, 0)),
        ],
        out_specs=pl.BlockSpec((1000, D), lambda i: (i, 0)),
        out_shape=jax.ShapeDtypeStruct((N, D), jnp.float32),
    )(x, W1, b1.reshape(1, D))


def _dis_block(deg_ref, pid):
    deg = deg_ref[0, :, 0:1] + deg_ref[1, :, 0:1] + 1.0
    dis = lax.rsqrt(deg)
    rows = pid * _RB + lax.broadcasted_iota(jnp.int32, (_RB, 1), 0)
    return jnp.where(rows < N, dis, 0.0)


def _scale0_kernel(deg_ref, h_ref, g_ref, r_ref):
    dis = _dis_block(deg_ref, pl.program_id(0))
    h = h_ref[...]
    g_ref[...] = h * dis
    r_ref[...] = jnp.maximum(h, 0.0)


def _scale0(deg_part, hp):
    return pl.pallas_call(
        _scale0_kernel,
        grid=(N_EXT // _RB,),
        in_specs=[
            pl.BlockSpec((NC, _RB, 16), lambda i: (0, i, 0)),
            pl.BlockSpec((_RB, D), lambda i: (i, 0)),
        ],
        out_specs=[
            pl.BlockSpec((_RB, D), lambda i: (i, 0)),
            pl.BlockSpec((_RB, D), lambda i: (i, 0)),
        ],
        out_shape=[
            jax.ShapeDtypeStruct((N_EXT, D), jnp.float32),
            jax.ShapeDtypeStruct((N_EXT, D), jnp.float32),
        ],
    )(deg_part, hp)


def _mid_kernel(deg_ref, acc_ref, g_ref, go_ref, r_ref):
    dis = _dis_block(deg_ref, pl.program_id(0))
    h = (acc_ref[0] + acc_ref[1] + g_ref[...]) * dis
    go_ref[...] = h * dis
    r_ref[...] = jnp.maximum(h, 0.0)


def _mid(deg_part, acc, g):
    return pl.pallas_call(
        _mid_kernel,
        grid=(N_EXT // _RB,),
        in_specs=[
            pl.BlockSpec((NC, _RB, 16), lambda i: (0, i, 0)),
            pl.BlockSpec((NC, _RB, D), lambda i: (0, i, 0)),
            pl.BlockSpec((_RB, D), lambda i: (i, 0)),
        ],
        out_specs=[
            pl.BlockSpec((_RB, D), lambda i: (i, 0)),
            pl.BlockSpec((_RB, D), lambda i: (i, 0)),
        ],
        out_shape=[
            jax.ShapeDtypeStruct((N_EXT, D), jnp.float32),
            jax.ShapeDtypeStruct((N_EXT, D), jnp.float32),
        ],
    )(deg_part, acc, g)


def _out_kernel(r0_ref, r1_ref, r2_ref, wa_ref, wb_ref, wc_ref, b_ref, o_ref):
    z = (
        jnp.dot(r0_ref[...], wa_ref[...], preferred_element_type=jnp.float32,
                precision=lax.Precision.HIGHEST)
        + jnp.dot(r1_ref[...], wb_ref[...], preferred_element_type=jnp.float32,
                  precision=lax.Precision.HIGHEST)
        + jnp.dot(r2_ref[...], wc_ref[...], preferred_element_type=jnp.float32,
                  precision=lax.Precision.HIGHEST)
        + b_ref[...]
    )
    m = jnp.max(z, axis=1, keepdims=True)
    e = jnp.exp(z - m)
    s = jnp.sum(e, axis=1, keepdims=True)
    o_ref[...] = z - m - jnp.log(s)


def _out(r0, r1, r2, W2, b2):
    wspec = pl.BlockSpec((D, NCLS), lambda i: (0, 0))
    rspec = pl.BlockSpec((1000, D), lambda i: (i, 0))
    return pl.pallas_call(
        _out_kernel,
        grid=(10,),
        in_specs=[rspec, rspec, rspec, wspec, wspec, wspec,
                  pl.BlockSpec((1, NCLS), lambda i: (0, 0))],
        out_specs=pl.BlockSpec((1000, NCLS), lambda i: (i, 0)),
        out_shape=jax.ShapeDtypeStruct((N, NCLS), jnp.float32),
    )(r0, r1, r2, W2[0:D], W2[D:2 * D], W2[2 * D:3 * D], b2.reshape(1, NCLS))


def kernel(x, edge_index, W1, b1, W2, b2):
    src = edge_index[0].astype(jnp.int32).reshape(NW, NE // NW)
    dst = edge_index[1].astype(jnp.int32).reshape(NW, NE // NW)
    pad = EPT - NE // NW
    src = jnp.pad(src, ((0, 0), (0, pad)), constant_values=N)
    dst = jnp.pad(dst, ((0, 0), (0, pad)), constant_values=N)

    deg_part = _deg_sc(dst)
    h0 = _lin1(x, W1, b1)
    hp0 = jnp.pad(h0, ((0, N_EXT - N), (0, 0)))
    g0, r0 = _scale0(deg_part, hp0)
    acc1 = _hop_sc(g0, src, dst)
    g1, r1 = _mid(deg_part, acc1, g0)
    acc2 = _hop_sc(g1, src, dst)
    _, r2 = _mid(deg_part, acc2, g1)
    return _out(r0, r1, r2, W2, b2)


# speedup vs baseline: 8.1593x; 8.1593x over previous
import jax
import jax.numpy as jnp
from jax import lax
from jax.experimental import pallas as pl
from jax.experimental.pallas import tpu as pltpu
from jax.experimental.pallas import tpu_sc as plsc

N = 10000
D = 128
NE = 320000
NCLS = 40
NC = 2
NS = 16
NW = NC * NS
N_EXT = 10240
EPT = NE // NW + 240
CH = 128
NCHUNK = EPT // CH
RPT = N_EXT // NS

_mesh = plsc.VectorSubcoreMesh(
    core_axis_name="c", subcore_axis_name="s", num_cores=NC, num_subcores=NS
)


def _row_indices():
    r = jnp.arange(N_EXT, dtype=jnp.int32).reshape(NS, RPT // CH, CH)
    return r


def _deg_body(dst_hbm, ridx_hbm, out_hbm, idx_v, ones_v, zbuf, ribuf, degacc):
    cid = lax.axis_index("c")
    sid = lax.axis_index("s")
    wid = cid * NS + sid

    @pl.loop(0, CH)
    def _(r):
        ones_v[r, :] = jnp.ones((16,), jnp.float32)
        zbuf[r, :] = jnp.zeros((16,), jnp.float32)

    @pl.loop(0, RPT // CH)
    def _(j):
        pltpu.sync_copy(ridx_hbm.at[sid, j], ribuf)
        pltpu.sync_copy(zbuf, degacc.at[ribuf])

    plsc.subcore_barrier()

    @pl.loop(0, NCHUNK)
    def _(c):
        pltpu.sync_copy(dst_hbm.at[wid, pl.ds(c * CH, CH)], idx_v)
        pltpu.sync_copy(ones_v, degacc.at[idx_v], add=True)

    plsc.subcore_barrier()

    @pl.loop(0, RPT // CH)
    def _(j):
        pltpu.sync_copy(ridx_hbm.at[sid, j], ribuf)
        pltpu.sync_copy(degacc.at[ribuf], zbuf)
        pltpu.sync_copy(zbuf, out_hbm.at[cid, pl.ds(sid * RPT + j * CH, CH)])


def _deg_sc(dst, ridx):
    return pl.kernel(
        _deg_body,
        out_type=jax.ShapeDtypeStruct((NC, N_EXT, 16), jnp.float32),
        mesh=_mesh,
        scratch_types=[
            pltpu.VMEM((CH,), jnp.int32),
            pltpu.VMEM((CH, 16), jnp.float32),
            pltpu.VMEM((CH, 16), jnp.float32),
            pltpu.VMEM((CH,), jnp.int32),
            pltpu.VMEM_SHARED((N_EXT, 16), jnp.float32),
        ],
    )(dst, ridx)


def _hop_body(g_hbm, src_hbm, dst_hbm, ridx_hbm, out_hbm,
              sidx, didx, rows, zbuf, ribuf, acc):
    cid = lax.axis_index("c")
    sid = lax.axis_index("s")
    wid = cid * NS + sid

    @pl.loop(0, CH)
    def _(r):
        @pl.loop(0, D, step=16)
        def _(c):
            zbuf[r, pl.ds(c, 16)] = jnp.zeros((16,), jnp.float32)

    @pl.loop(0, RPT // CH)
    def _(j):
        pltpu.sync_copy(ridx_hbm.at[sid, j], ribuf)
        pltpu.sync_copy(zbuf, acc.at[ribuf])

    plsc.subcore_barrier()

    @pl.loop(0, NCHUNK)
    def _(c):
        pltpu.sync_copy(src_hbm.at[wid, pl.ds(c * CH, CH)], sidx)
        pltpu.sync_copy(g_hbm.at[sidx], rows)
        pltpu.sync_copy(dst_hbm.at[wid, pl.ds(c * CH, CH)], didx)
        pltpu.sync_copy(rows, acc.at[didx], add=True)

    plsc.subcore_barrier()

    @pl.loop(0, RPT // CH)
    def _(j):
        pltpu.sync_copy(ridx_hbm.at[sid, j], ribuf)
        pltpu.sync_copy(acc.at[ribuf], rows)
        pltpu.sync_copy(rows, out_hbm.at[cid, pl.ds(sid * RPT + j * CH, CH)])


def _hop_sc(g, src, dst, ridx):
    return pl.kernel(
        _hop_body,
        out_type=jax.ShapeDtypeStruct((NC, N_EXT, D), jnp.float32),
        mesh=_mesh,
        scratch_types=[
            pltpu.VMEM((CH,), jnp.int32),
            pltpu.VMEM((CH,), jnp.int32),
            pltpu.VMEM((CH, D), jnp.float32),
            pltpu.VMEM((CH, D), jnp.float32),
            pltpu.VMEM((CH,), jnp.int32),
            pltpu.VMEM_SHARED((N_EXT, D), jnp.float32),
        ],
    )(g, src, dst, ridx)


_RB = 1024


def _lin1_kernel(x_ref, w_ref, b_ref, o_ref):
    o_ref[...] = (
        jnp.dot(
            x_ref[...],
            w_ref[...],
            preferred_element_type=jnp.float32,
            precision=lax.Precision.HIGHEST,
        )
        + b_ref[...]
    )


def _lin1(x, W1, b1):
    return pl.pallas_call(
        _lin1_kernel,
        grid=(10,),
        in_specs=[
            pl.BlockSpec((1000, D), lambda i: (i, 0)),
            pl.BlockSpec((D, D), lambda i: (0, 0)),
            pl.BlockSpec((1, D), lambda i: (0, 0)),
        ],
        out_specs=pl.BlockSpec((1000, D), lambda i: (i, 0)),
        out_shape=jax.ShapeDtypeStruct((N, D), jnp.float32),
    )(x, W1, b1.reshape(1, D))


def _dis_block(deg_ref, pid):
    deg = deg_ref[0, :, 0:1] + deg_ref[1, :, 0:1] + 1.0
    dis = lax.rsqrt(deg)
    rows = pid * _RB + lax.broadcasted_iota(jnp.int32, (_RB, 1), 0)
    return jnp.where(rows < N, dis, 0.0)


def _scale0_kernel(deg_ref, h_ref, g_ref, r_ref):
    dis = _dis_block(deg_ref, pl.program_id(0))
    h = h_ref[...]
    g_ref[...] = h * dis
    r_ref[...] = jnp.maximum(h, 0.0)


def _scale0(deg_part, hp):
    return pl.pallas_call(
        _scale0_kernel,
        grid=(N_EXT // _RB,),
        in_specs=[
            pl.BlockSpec((NC, _RB, 16), lambda i: (0, i, 0)),
            pl.BlockSpec((_RB, D), lambda i: (i, 0)),
        ],
        out_specs=[
            pl.BlockSpec((_RB, D), lambda i: (i, 0)),
            pl.BlockSpec((_RB, D), lambda i: (i, 0)),
        ],
        out_shape=[
            jax.ShapeDtypeStruct((N_EXT, D), jnp.float32),
            jax.ShapeDtypeStruct((N_EXT, D), jnp.float32),
        ],
    )(deg_part, hp)


def _mid_kernel(deg_ref, acc_ref, g_ref, go_ref, r_ref):
    dis = _dis_block(deg_ref, pl.program_id(0))
    h = (acc_ref[0] + acc_ref[1] + g_ref[...]) * dis
    go_ref[...] = h * dis
    r_ref[...] = jnp.maximum(h, 0.0)


def _mid(deg_part, acc, g):
    return pl.pallas_call(
        _mid_kernel,
        grid=(N_EXT // _RB,),
        in_specs=[
            pl.BlockSpec((NC, _RB, 16), lambda i: (0, i, 0)),
            pl.BlockSpec((NC, _RB, D), lambda i: (0, i, 0)),
            pl.BlockSpec((_RB, D), lambda i: (i, 0)),
        ],
        out_specs=[
            pl.BlockSpec((_RB, D), lambda i: (i, 0)),
            pl.BlockSpec((_RB, D), lambda i: (i, 0)),
        ],
        out_shape=[
            jax.ShapeDtypeStruct((N_EXT, D), jnp.float32),
            jax.ShapeDtypeStruct((N_EXT, D), jnp.float32),
        ],
    )(deg_part, acc, g)


def _out_kernel(r0_ref, r1_ref, r2_ref, wa_ref, wb_ref, wc_ref, b_ref, o_ref):
    z = (
        jnp.dot(r0_ref[...], wa_ref[...], preferred_element_type=jnp.float32,
                precision=lax.Precision.HIGHEST)
        + jnp.dot(r1_ref[...], wb_ref[...], preferred_element_type=jnp.float32,
                  precision=lax.Precision.HIGHEST)
        + jnp.dot(r2_ref[...], wc_ref[...], preferred_element_type=jnp.float32,
                  precision=lax.Precision.HIGHEST)
        + b_ref[...]
    )
    m = jnp.max(z, axis=1, keepdims=True)
    e = jnp.exp(z - m)
    s = jnp.sum(e, axis=1, keepdims=True)
    o_ref[...] = z - m - jnp.log(s)


def _out(r0, r1, r2, W2, b2):
    wspec = pl.BlockSpec((D, NCLS), lambda i: (0, 0))
    rspec = pl.BlockSpec((1000, D), lambda i: (i, 0))
    return pl.pallas_call(
        _out_kernel,
        grid=(10,),
        in_specs=[rspec, rspec, rspec, wspec, wspec, wspec,
                  pl.BlockSpec((1, NCLS), lambda i: (0, 0))],
        out_specs=pl.BlockSpec((1000, NCLS), lambda i: (i, 0)),
        out_shape=jax.ShapeDtypeStruct((N, NCLS), jnp.float32),
    )(r0, r1, r2, W2[0:D], W2[D:2 * D], W2[2 * D:3 * D], b2.reshape(1, NCLS))


def kernel(x, edge_index, W1, b1, W2, b2):
    src = edge_index[0].astype(jnp.int32).reshape(NW, NE // NW)
    dst = edge_index[1].astype(jnp.int32).reshape(NW, NE // NW)
    pad = EPT - NE // NW
    src = jnp.pad(src, ((0, 0), (0, pad)), constant_values=N)
    dst = jnp.pad(dst, ((0, 0), (0, pad)), constant_values=N)

    ridx = _row_indices()
    deg_part = _deg_sc(dst, ridx)
    h0 = _lin1(x, W1, b1)
    hp0 = jnp.pad(h0, ((0, N_EXT - N), (0, 0)))
    g0, r0 = _scale0(deg_part, hp0)
    acc1 = _hop_sc(g0, src, dst, ridx)
    g1, r1 = _mid(deg_part, acc1, g0)
    acc2 = _hop_sc(g1, src, dst, ridx)
    _, r2 = _mid(deg_part, acc2, g1)
    return _out(r0, r1, r2, W2, b2)


# speedup vs baseline: 10.4607x; 1.2821x over previous
import jax
import jax.numpy as jnp
from jax import lax
from jax.experimental import pallas as pl
from jax.experimental.pallas import tpu as pltpu
from jax.experimental.pallas import tpu_sc as plsc

N = 10000
D = 128
NE = 320000
NCLS = 40
NC = 2
NS = 16
NW = NC * NS
N_EXT = 10240
EPT = NE // NW + 240
CH = 128
NCHUNK = EPT // CH
RPT = N_EXT // NS

_mesh = plsc.VectorSubcoreMesh(
    core_axis_name="c", subcore_axis_name="s", num_cores=NC, num_subcores=NS
)


def _row_indices():
    r = jnp.arange(N_EXT, dtype=jnp.int32).reshape(NS, RPT // CH, CH)
    return r


NB = 5
NBLK = NCHUNK // NB
NRO = RPT // CH


def _deg_body(dst_hbm, ridx_hbm, out_hbm, didx_all, ones_v, zbuf, ribuf, ssem,
              degacc):
    cid = lax.axis_index("c")
    sid = lax.axis_index("s")
    wid = cid * NS + sid

    pltpu.sync_copy(dst_hbm.at[wid], didx_all)
    pltpu.sync_copy(ridx_hbm.at[sid], ribuf)

    @pl.loop(0, CH)
    def _(r):
        ones_v[r, :] = jnp.ones((16,), jnp.float32)
        zbuf[r, :] = jnp.zeros((16,), jnp.float32)

    @pl.loop(0, NRO)
    def _(j):
        pltpu.sync_copy(zbuf, degacc.at[ribuf.at[j]])

    plsc.subcore_barrier()

    @pl.loop(0, NCHUNK)
    def _(c):
        pltpu.sync_copy(ones_v, degacc.at[didx_all.at[c]], add=True)

    plsc.subcore_barrier()

    @pl.loop(0, NRO)
    def _(j):
        pltpu.sync_copy(degacc.at[ribuf.at[j]], zbuf)
        pltpu.sync_copy(zbuf, out_hbm.at[cid, pl.ds(sid * RPT + j * CH, CH)])


def _deg_sc(dst, ridx):
    return pl.kernel(
        _deg_body,
        out_type=jax.ShapeDtypeStruct((NC, N_EXT, 16), jnp.float32),
        mesh=_mesh,
        scratch_types=[
            pltpu.VMEM((NCHUNK, CH), jnp.int32),
            pltpu.VMEM((CH, 16), jnp.float32),
            pltpu.VMEM((CH, 16), jnp.float32),
            pltpu.VMEM((NRO, CH), jnp.int32),
            pltpu.SemaphoreType.DMA,
            pltpu.VMEM_SHARED((N_EXT, 16), jnp.float32),
        ],
    )(dst, ridx)


def _hop_body(g_hbm, e_hbm, ridx_hbm, out_hbm,
              idxb0, idxb1, rows0, rows1, ribuf, gsem, isem, acc):
    cid = lax.axis_index("c")
    sid = lax.axis_index("s")
    wid = cid * NS + sid
    rows = (rows0, rows1)
    idxb = (idxb0, idxb1)

    pltpu.sync_copy(ridx_hbm.at[sid], ribuf)

    @pl.loop(0, CH)
    def _(r):
        @pl.loop(0, D, step=16)
        def _(c):
            rows0[r, pl.ds(c, 16)] = jnp.zeros((16,), jnp.float32)

    @pl.loop(0, NRO)
    def _(j):
        pltpu.sync_copy(rows0, acc.at[ribuf.at[j]])

    plsc.subcore_barrier()

    pltpu.sync_copy(e_hbm.at[wid, 0], idxb0)
    pltpu.async_copy(g_hbm.at[idxb0.at[0]], rows0, gsem.at[0])
    pltpu.async_copy(e_hbm.at[wid, 1], idxb1, isem.at[1])

    @pl.loop(0, NCHUNK // 2)
    def _(j):
        for b in range(2):
            c = 2 * j + b
            bn = 1 - b
            pltpu.make_async_copy(e_hbm.at[wid, 0], idxb[bn], isem.at[bn]).wait()
            pltpu.async_copy(g_hbm.at[idxb[bn].at[0]], rows[bn], gsem.at[bn])
            pltpu.make_async_copy(
                g_hbm.at[idxb[b].at[0]], rows[b], gsem.at[b]).wait()
            pltpu.sync_copy(rows[b], acc.at[idxb[b].at[1]], add=True)
            pltpu.async_copy(
                e_hbm.at[wid, lax.rem(c + 2, NCHUNK)], idxb[b], isem.at[b])

    pltpu.make_async_copy(g_hbm.at[idxb0.at[0]], rows0, gsem.at[0]).wait()
    pltpu.make_async_copy(e_hbm.at[wid, 0], idxb1, isem.at[1]).wait()

    plsc.subcore_barrier()

    @pl.loop(0, NRO)
    def _(j):
        pltpu.sync_copy(acc.at[ribuf.at[j]], rows0)
        pltpu.sync_copy(rows0,
                        out_hbm.at[cid, pl.ds(sid * RPT + j * CH, CH)])


def _hop_sc(g, e, ridx):
    return pl.kernel(
        _hop_body,
        out_type=jax.ShapeDtypeStruct((NC, N_EXT, D), jnp.float32),
        mesh=_mesh,
        scratch_types=[
            pltpu.VMEM((2, CH), jnp.int32),
            pltpu.VMEM((2, CH), jnp.int32),
            pltpu.VMEM((CH, D), jnp.float32),
            pltpu.VMEM((CH, D), jnp.float32),
            pltpu.VMEM((NRO, CH), jnp.int32),
            pltpu.SemaphoreType.DMA((2,)),
            pltpu.SemaphoreType.DMA((2,)),
            pltpu.VMEM_SHARED((N_EXT, D), jnp.float32),
        ],
    )(g, e, ridx)


_RB = 1024


def _lin1_kernel(x_ref, w_ref, b_ref, o_ref):
    o_ref[...] = (
        jnp.dot(
            x_ref[...],
            w_ref[...],
            preferred_element_type=jnp.float32,
            precision=lax.Precision.HIGHEST,
        )
        + b_ref[...]
    )


def _lin1(x, W1, b1):
    return pl.pallas_call(
        _lin1_kernel,
        grid=(10,),
        in_specs=[
            pl.BlockSpec((1000, D), lambda i: (i, 0)),
            pl.BlockSpec((D, D), lambda i: (0, 0)),
            pl.BlockSpec((1, D), lambda i: (0, 0)),
        ],
        out_specs=pl.BlockSpec((1000, D), lambda i: (i, 0)),
        out_shape=jax.ShapeDtypeStruct((N, D), jnp.float32),
    )(x, W1, b1.reshape(1, D))


def _dis_block(deg_ref, pid):
    deg = deg_ref[0, :, 0:1] + deg_ref[1, :, 0:1] + 1.0
    dis = lax.rsqrt(deg)
    rows = pid * _RB + lax.broadcasted_iota(jnp.int32, (_RB, 1), 0)
    return jnp.where(rows < N, dis, 0.0)


def _scale0_kernel(deg_ref, h_ref, g_ref, r_ref):
    dis = _dis_block(deg_ref, pl.program_id(0))
    h = h_ref[...]
    g_ref[...] = h * dis
    r_ref[...] = jnp.maximum(h, 0.0)


def _scale0(deg_part, hp):
    return pl.pallas_call(
        _scale0_kernel,
        grid=(N_EXT // _RB,),
        in_specs=[
            pl.BlockSpec((NC, _RB, 16), lambda i: (0, i, 0)),
            pl.BlockSpec((_RB, D), lambda i: (i, 0)),
        ],
        out_specs=[
            pl.BlockSpec((_RB, D), lambda i: (i, 0)),
            pl.BlockSpec((_RB, D), lambda i: (i, 0)),
        ],
        out_shape=[
            jax.ShapeDtypeStruct((N_EXT, D), jnp.float32),
            jax.ShapeDtypeStruct((N_EXT, D), jnp.float32),
        ],
    )(deg_part, hp)


def _mid_kernel(deg_ref, acc_ref, g_ref, go_ref, r_ref):
    dis = _dis_block(deg_ref, pl.program_id(0))
    h = (acc_ref[0] + acc_ref[1] + g_ref[...]) * dis
    go_ref[...] = h * dis
    r_ref[...] = jnp.maximum(h, 0.0)


def _mid(deg_part, acc, g):
    return pl.pallas_call(
        _mid_kernel,
        grid=(N_EXT // _RB,),
        in_specs=[
            pl.BlockSpec((NC, _RB, 16), lambda i: (0, i, 0)),
            pl.BlockSpec((NC, _RB, D), lambda i: (0, i, 0)),
            pl.BlockSpec((_RB, D), lambda i: (i, 0)),
        ],
        out_specs=[
            pl.BlockSpec((_RB, D), lambda i: (i, 0)),
            pl.BlockSpec((_RB, D), lambda i: (i, 0)),
        ],
        out_shape=[
            jax.ShapeDtypeStruct((N_EXT, D), jnp.float32),
            jax.ShapeDtypeStruct((N_EXT, D), jnp.float32),
        ],
    )(deg_part, acc, g)


def _out_kernel(r0_ref, r1_ref, r2_ref, wa_ref, wb_ref, wc_ref, b_ref, o_ref):
    z = (
        jnp.dot(r0_ref[...], wa_ref[...], preferred_element_type=jnp.float32,
                precision=lax.Precision.HIGHEST)
        + jnp.dot(r1_ref[...], wb_ref[...], preferred_element_type=jnp.float32,
                  precision=lax.Precision.HIGHEST)
        + jnp.dot(r2_ref[...], wc_ref[...], preferred_element_type=jnp.float32,
                  precision=lax.Precision.HIGHEST)
        + b_ref[...]
    )
    m = jnp.max(z, axis=1, keepdims=True)
    e = jnp.exp(z - m)
    s = jnp.sum(e, axis=1, keepdims=True)
    o_ref[...] = z - m - jnp.log(s)


def _out(r0, r1, r2, W2, b2):
    wspec = pl.BlockSpec((D, NCLS), lambda i: (0, 0))
    rspec = pl.BlockSpec((1000, D), lambda i: (i, 0))
    return pl.pallas_call(
        _out_kernel,
        grid=(10,),
        in_specs=[rspec, rspec, rspec, wspec, wspec, wspec,
                  pl.BlockSpec((1, NCLS), lambda i: (0, 0))],
        out_specs=pl.BlockSpec((1000, NCLS), lambda i: (i, 0)),
        out_shape=jax.ShapeDtypeStruct((N, NCLS), jnp.float32),
    )(r0, r1, r2, W2[0:D], W2[D:2 * D], W2[2 * D:3 * D], b2.reshape(1, NCLS))


def kernel(x, edge_index, W1, b1, W2, b2):
    src = edge_index[0].astype(jnp.int32).reshape(NW, NE // NW)
    dst = edge_index[1].astype(jnp.int32).reshape(NW, NE // NW)
    pad = EPT - NE // NW
    src = jnp.pad(src, ((0, 0), (0, pad)), constant_values=N).reshape(
        NW, NCHUNK, CH)
    dst = jnp.pad(dst, ((0, 0), (0, pad)), constant_values=N).reshape(
        NW, NCHUNK, CH)
    e = jnp.stack([src, dst], axis=2)

    ridx = _row_indices()
    deg_part = _deg_sc(dst, ridx)
    h0 = _lin1(x, W1, b1)
    hp0 = jnp.pad(h0, ((0, N_EXT - N), (0, 0)))
    g0, r0 = _scale0(deg_part, hp0)
    acc1 = _hop_sc(g0, e, ridx)
    g1, r1 = _mid(deg_part, acc1, g0)
    acc2 = _hop_sc(g1, e, ridx)
    _, r2 = _mid(deg_part, acc2, g1)
    return _out(r0, r1, r2, W2, b2)


# speedup vs baseline: 10.8445x; 1.0367x over previous
import jax
import jax.numpy as jnp
from jax import lax
from jax.experimental import pallas as pl
from jax.experimental.pallas import tpu as pltpu
from jax.experimental.pallas import tpu_sc as plsc

N = 10000
D = 128
NE = 320000
NCLS = 40
NC = 2
NS = 16
NW = NC * NS
N_EXT = 10240
EPT = NE // NW + 240
CH = 128
NCHUNK = EPT // CH
RPT = N_EXT // NS

_mesh = plsc.VectorSubcoreMesh(
    core_axis_name="c", subcore_axis_name="s", num_cores=NC, num_subcores=NS
)


def _row_indices():
    r = jnp.arange(N_EXT, dtype=jnp.int32).reshape(NS, RPT // CH, CH)
    return r


NB = 5
NBLK = NCHUNK // NB
NRO = RPT // CH


def _deg_body(dst_hbm, ridx_hbm, out_hbm, didx_all, ones_v, zbuf, ribuf, ssem,
              degacc):
    cid = lax.axis_index("c")
    sid = lax.axis_index("s")
    wid = cid * NS + sid

    pltpu.sync_copy(dst_hbm.at[wid], didx_all)
    pltpu.sync_copy(ridx_hbm.at[sid], ribuf)

    @pl.loop(0, CH)
    def _(r):
        ones_v[r, :] = jnp.ones((16,), jnp.float32)
        zbuf[r, :] = jnp.zeros((16,), jnp.float32)

    @pl.loop(0, NRO)
    def _(j):
        pltpu.sync_copy(zbuf, degacc.at[ribuf.at[j]])

    plsc.subcore_barrier()

    @pl.loop(0, NCHUNK)
    def _(c):
        pltpu.sync_copy(ones_v, degacc.at[didx_all.at[c]], add=True)

    plsc.subcore_barrier()

    @pl.loop(0, NRO)
    def _(j):
        pltpu.sync_copy(degacc.at[ribuf.at[j]], zbuf)
        pltpu.sync_copy(zbuf, out_hbm.at[cid, pl.ds(sid * RPT + j * CH, CH)])


def _deg_sc(dst, ridx):
    return pl.kernel(
        _deg_body,
        out_type=jax.ShapeDtypeStruct((NC, N_EXT, 16), jnp.float32),
        mesh=_mesh,
        scratch_types=[
            pltpu.VMEM((NCHUNK, CH), jnp.int32),
            pltpu.VMEM((CH, 16), jnp.float32),
            pltpu.VMEM((CH, 16), jnp.float32),
            pltpu.VMEM((NRO, CH), jnp.int32),
            pltpu.SemaphoreType.DMA,
            pltpu.VMEM_SHARED((N_EXT, 16), jnp.float32),
        ],
    )(dst, ridx)


def _hop_body(g_hbm, e_hbm, ridx_hbm, out_hbm,
              idxb0, idxb1, idxb2, idxb3, rows0, rows1, ribuf,
              gsem, isem, ssem, acc):
    cid = lax.axis_index("c")
    sid = lax.axis_index("s")
    wid = cid * NS + sid
    rows = (rows0, rows1)
    idxb = (idxb0, idxb1, idxb2, idxb3)

    pltpu.sync_copy(ridx_hbm.at[sid], ribuf)

    @pl.loop(0, CH)
    def _(r):
        @pl.loop(0, D, step=16)
        def _(c):
            rows0[r, pl.ds(c, 16)] = jnp.zeros((16,), jnp.float32)
            rows1[r, pl.ds(c, 16)] = jnp.zeros((16,), jnp.float32)

    @pl.loop(0, NRO)
    def _(j):
        pltpu.sync_copy(rows0, acc.at[ribuf.at[j]])

    plsc.subcore_barrier()

    pltpu.sync_copy(e_hbm.at[wid, 0], idxb[0])
    for k in range(1, 3):
        pltpu.async_copy(e_hbm.at[wid, k], idxb[k], isem.at[k])
    pltpu.async_copy(g_hbm.at[idxb[0].at[0]], rows0, gsem.at[0])
    pltpu.async_copy(rows1, acc.at[ribuf.at[0]], ssem.at[1], add=True)

    @pl.loop(0, NCHUNK // 4)
    def _(q):
        for k in range(4):
            c = 4 * q + k
            b = k % 2
            bn = 1 - b
            s1 = (k + 1) % 4
            s3 = (k + 3) % 4
            pltpu.make_async_copy(
                rows[bn], acc.at[idxb[s3].at[1]], ssem.at[bn]).wait()
            pltpu.async_copy(
                e_hbm.at[wid, lax.rem(c + 3, NCHUNK)], idxb[s3], isem.at[s3])
            pltpu.make_async_copy(
                e_hbm.at[wid, 0], idxb[s1], isem.at[s1]).wait()
            pltpu.async_copy(g_hbm.at[idxb[s1].at[0]], rows[bn], gsem.at[bn])
            pltpu.make_async_copy(
                g_hbm.at[idxb[k].at[0]], rows[b], gsem.at[b]).wait()
            pltpu.async_copy(rows[b], acc.at[idxb[k].at[1]], ssem.at[b],
                             add=True)

    pltpu.make_async_copy(e_hbm.at[wid, 0], idxb[1], isem.at[1]).wait()
    pltpu.make_async_copy(e_hbm.at[wid, 0], idxb[2], isem.at[2]).wait()
    pltpu.make_async_copy(g_hbm.at[idxb[0].at[0]], rows0, gsem.at[0]).wait()
    pltpu.make_async_copy(rows1, acc.at[idxb[3].at[1]], ssem.at[1]).wait()

    plsc.subcore_barrier()

    @pl.loop(0, NRO)
    def _(j):
        pltpu.sync_copy(acc.at[ribuf.at[j]], rows0)
        pltpu.sync_copy(rows0,
                        out_hbm.at[cid, pl.ds(sid * RPT + j * CH, CH)])


def _hop_sc(g, e, ridx):
    return pl.kernel(
        _hop_body,
        out_type=jax.ShapeDtypeStruct((NC, N_EXT, D), jnp.float32),
        mesh=_mesh,
        scratch_types=[
            pltpu.VMEM((2, CH), jnp.int32),
            pltpu.VMEM((2, CH), jnp.int32),
            pltpu.VMEM((2, CH), jnp.int32),
            pltpu.VMEM((2, CH), jnp.int32),
            pltpu.VMEM((CH, D), jnp.float32),
            pltpu.VMEM((CH, D), jnp.float32),
            pltpu.VMEM((NRO, CH), jnp.int32),
            pltpu.SemaphoreType.DMA((2,)),
            pltpu.SemaphoreType.DMA((4,)),
            pltpu.SemaphoreType.DMA((2,)),
            pltpu.VMEM_SHARED((N_EXT, D), jnp.float32),
        ],
    )(g, e, ridx)


_RB = 1024


def _lin1_kernel(x_ref, w_ref, b_ref, o_ref):
    o_ref[...] = (
        jnp.dot(
            x_ref[...],
            w_ref[...],
            preferred_element_type=jnp.float32,
            precision=lax.Precision.HIGHEST,
        )
        + b_ref[...]
    )


def _lin1(x, W1, b1):
    return pl.pallas_call(
        _lin1_kernel,
        grid=(10,),
        in_specs=[
            pl.BlockSpec((1000, D), lambda i: (i, 0)),
            pl.BlockSpec((D, D), lambda i: (0, 0)),
            pl.BlockSpec((1, D), lambda i: (0, 0)),
        ],
        out_specs=pl.BlockSpec((1000, D), lambda i: (i, 0)),
        out_shape=jax.ShapeDtypeStruct((N, D), jnp.float32),
    )(x, W1, b1.reshape(1, D))


def _dis_block(deg_ref, pid):
    deg = deg_ref[0, :, 0:1] + deg_ref[1, :, 0:1] + 1.0
    dis = lax.rsqrt(deg)
    rows = pid * _RB + lax.broadcasted_iota(jnp.int32, (_RB, 1), 0)
    return jnp.where(rows < N, dis, 0.0)


def _scale0_kernel(deg_ref, h_ref, g_ref, r_ref):
    dis = _dis_block(deg_ref, pl.program_id(0))
    h = h_ref[...]
    g_ref[...] = h * dis
    r_ref[...] = jnp.maximum(h, 0.0)


def _scale0(deg_part, hp):
    return pl.pallas_call(
        _scale0_kernel,
        grid=(N_EXT // _RB,),
        in_specs=[
            pl.BlockSpec((NC, _RB, 16), lambda i: (0, i, 0)),
            pl.BlockSpec((_RB, D), lambda i: (i, 0)),
        ],
        out_specs=[
            pl.BlockSpec((_RB, D), lambda i: (i, 0)),
            pl.BlockSpec((_RB, D), lambda i: (i, 0)),
        ],
        out_shape=[
            jax.ShapeDtypeStruct((N_EXT, D), jnp.float32),
            jax.ShapeDtypeStruct((N_EXT, D), jnp.float32),
        ],
    )(deg_part, hp)


def _mid_kernel(deg_ref, acc_ref, g_ref, go_ref, r_ref):
    dis = _dis_block(deg_ref, pl.program_id(0))
    h = (acc_ref[0] + acc_ref[1] + g_ref[...]) * dis
    go_ref[...] = h * dis
    r_ref[...] = jnp.maximum(h, 0.0)


def _mid(deg_part, acc, g):
    return pl.pallas_call(
        _mid_kernel,
        grid=(N_EXT // _RB,),
        in_specs=[
            pl.BlockSpec((NC, _RB, 16), lambda i: (0, i, 0)),
            pl.BlockSpec((NC, _RB, D), lambda i: (0, i, 0)),
            pl.BlockSpec((_RB, D), lambda i: (i, 0)),
        ],
        out_specs=[
            pl.BlockSpec((_RB, D), lambda i: (i, 0)),
            pl.BlockSpec((_RB, D), lambda i: (i, 0)),
        ],
        out_shape=[
            jax.ShapeDtypeStruct((N_EXT, D), jnp.float32),
            jax.ShapeDtypeStruct((N_EXT, D), jnp.float32),
        ],
    )(deg_part, acc, g)


def _out_kernel(r0_ref, r1_ref, r2_ref, wa_ref, wb_ref, wc_ref, b_ref, o_ref):
    z = (
        jnp.dot(r0_ref[...], wa_ref[...], preferred_element_type=jnp.float32,
                precision=lax.Precision.HIGHEST)
        + jnp.dot(r1_ref[...], wb_ref[...], preferred_element_type=jnp.float32,
                  precision=lax.Precision.HIGHEST)
        + jnp.dot(r2_ref[...], wc_ref[...], preferred_element_type=jnp.float32,
                  precision=lax.Precision.HIGHEST)
        + b_ref[...]
    )
    m = jnp.max(z, axis=1, keepdims=True)
    e = jnp.exp(z - m)
    s = jnp.sum(e, axis=1, keepdims=True)
    o_ref[...] = z - m - jnp.log(s)


def _out(r0, r1, r2, W2, b2):
    wspec = pl.BlockSpec((D, NCLS), lambda i: (0, 0))
    rspec = pl.BlockSpec((1000, D), lambda i: (i, 0))
    return pl.pallas_call(
        _out_kernel,
        grid=(10,),
        in_specs=[rspec, rspec, rspec, wspec, wspec, wspec,
                  pl.BlockSpec((1, NCLS), lambda i: (0, 0))],
        out_specs=pl.BlockSpec((1000, NCLS), lambda i: (i, 0)),
        out_shape=jax.ShapeDtypeStruct((N, NCLS), jnp.float32),
    )(r0, r1, r2, W2[0:D], W2[D:2 * D], W2[2 * D:3 * D], b2.reshape(1, NCLS))


def kernel(x, edge_index, W1, b1, W2, b2):
    src = edge_index[0].astype(jnp.int32).reshape(NW, NE // NW)
    dst = edge_index[1].astype(jnp.int32).reshape(NW, NE // NW)
    pad = EPT - NE // NW
    src = jnp.pad(src, ((0, 0), (0, pad)), constant_values=N).reshape(
        NW, NCHUNK, CH)
    dst = jnp.pad(dst, ((0, 0), (0, pad)), constant_values=N).reshape(
        NW, NCHUNK, CH)
    e = jnp.stack([src, dst], axis=2)

    ridx = _row_indices()
    deg_part = _deg_sc(dst, ridx)
    h0 = _lin1(x, W1, b1)
    hp0 = jnp.pad(h0, ((0, N_EXT - N), (0, 0)))
    g0, r0 = _scale0(deg_part, hp0)
    acc1 = _hop_sc(g0, e, ridx)
    g1, r1 = _mid(deg_part, acc1, g0)
    acc2 = _hop_sc(g1, e, ridx)
    _, r2 = _mid(deg_part, acc2, g1)
    return _out(r0, r1, r2, W2, b2)
